# Initial kernel scaffold; baseline (speedup 1.0000x reference)
import jax
import jax.numpy as jnp
from jax.experimental import pallas as pl


def kernel(unique_node_ids, unique_messages, timestamps, net_transaction_amounts, memory, last_update, W_cat, b_cat, W_ih, W_hh, b_ih, b_hh):
    raise NotImplementedError("write your pallas kernel here")


# speedup vs baseline: 1.1543x; 1.1543x over previous
import functools

import jax
import jax.numpy as jnp
from jax import lax
from jax.experimental import pallas as pl
from jax.experimental.pallas import tpu as pltpu

N_NODES = 100000
D = 128
B = 16384
BLK = 1024


def _gru_body(m_ref, amt_ref, h_ref, wm_ref, wa_ref, bc_ref,
              wih_ref, whh_ref, bih_ref, bhh_ref, out_ref):
    m = m_ref[:]
    h = h_ref[:]
    x = (jnp.dot(m, wm_ref[:], preferred_element_type=jnp.float32)
         + amt_ref[:] * wa_ref[:] + bc_ref[:])
    gi = jnp.dot(x, wih_ref[:], preferred_element_type=jnp.float32) + bih_ref[:]
    gh = jnp.dot(h, whh_ref[:], preferred_element_type=jnp.float32) + bhh_ref[:]
    r = jax.nn.sigmoid(gi[:, :D] + gh[:, :D])
    z = jax.nn.sigmoid(gi[:, D:2 * D] + gh[:, D:2 * D])
    n = jnp.tanh(gi[:, 2 * D:] + r * gh[:, 2 * D:])
    out_ref[:] = (1.0 - z) * n + z * h


def _gru_new_h(messages, amounts, h, W_cat, b_cat, W_ih, W_hh, b_ih, b_hh):
    wm = W_cat[:, :D].T
    wa = W_cat[:, D].reshape(1, D)
    bc = b_cat.reshape(1, D)
    wih = W_ih.T
    whh = W_hh.T
    bih = b_ih.reshape(1, 3 * D)
    bhh = b_hh.reshape(1, 3 * D)
    amt = amounts.reshape(B, 1)
    grid = (B // BLK,)
    blk_rows = lambda i: (i, 0)
    fixed = lambda i: (0, 0)
    return pl.pallas_call(
        _gru_body,
        grid=grid,
        in_specs=[
            pl.BlockSpec((BLK, D), blk_rows),
            pl.BlockSpec((BLK, 1), blk_rows),
            pl.BlockSpec((BLK, D), blk_rows),
            pl.BlockSpec((D, D), fixed),
            pl.BlockSpec((1, D), fixed),
            pl.BlockSpec((1, D), fixed),
            pl.BlockSpec((D, 3 * D), fixed),
            pl.BlockSpec((D, 3 * D), fixed),
            pl.BlockSpec((1, 3 * D), fixed),
            pl.BlockSpec((1, 3 * D), fixed),
        ],
        out_specs=pl.BlockSpec((BLK, D), blk_rows),
        out_shape=jax.ShapeDtypeStruct((B, D), jnp.float32),
    )(messages, amt, h, wm, wa, bc, wih, whh, bih, bhh)


def kernel(unique_node_ids, unique_messages, timestamps, net_transaction_amounts,
           memory, last_update, W_cat, b_cat, W_ih, W_hh, b_ih, b_hh):
    ids = unique_node_ids
    h = memory[ids]
    new_h = _gru_new_h(unique_messages, net_transaction_amounts, h,
                       W_cat, b_cat, W_ih, W_hh, b_ih, b_hh)
    updated_memory = memory.at[ids].set(new_h)
    updated_last_update = last_update.at[ids].set(timestamps)
    return (updated_memory, updated_last_update)
